# baseline (speedup 1.0000x reference)
import jax
import jax.numpy as jnp
from jax.experimental import pallas as pl
from jax.experimental.pallas import tpu as pltpu

_R, _C = 500, 20000
_WIDTHS = [2048] * 9 + [1024]
_BUFW = sum(_WIDTHS)
_TAILW = _C - _BUFW
_NFULL = len(_WIDTHS)
_NCH = _NFULL + 1
_OFS = [sum(_WIDTHS[:k]) for k in range(_NFULL)] + [_BUFW]

_LOW = 0.3
_HIGH = 0.7


def _body(x_hbm, out_ref, buf, tail, cmax_ref, cam_ref, rmax_ref, sems):
    def chunk_src(k):
        ofs = _OFS[k]
        if k < _NFULL:
            w = _WIDTHS[k]
            return ofs, w, buf.at[:, pl.ds(ofs, w)]
        return ofs, _TAILW, tail.at[:, :]

    copies = []
    for k in range(_NCH):
        ofs, w, dst = chunk_src(k)
        cp = pltpu.make_async_copy(x_hbm.at[:, pl.ds(ofs, w)], dst, sems.at[k])
        cp.start()
        copies.append(cp)

    def chunk_blk(k):
        ofs, w, _ = chunk_src(k)
        if k < _NFULL:
            return ofs, w, buf[:, pl.ds(ofs, w)]
        return ofs, w, tail[:, :]

    for k in range(_NCH):
        copies[k].wait()
        ofs, w, blk = chunk_blk(k)
        part_rm = jnp.max(blk, axis=1, keepdims=True)
        if k == 0:
            rmax_ref[...] = part_rm
        else:
            rmax_ref[...] = jnp.maximum(rmax_ref[...], part_rm)
        cmax = jnp.max(blk, axis=0)
        rows = jax.lax.broadcasted_iota(jnp.int32, (_R, w), 0)
        cam = jnp.min(jnp.where(blk == cmax[None, :], rows, _R), axis=0)
        cmax_ref[0, pl.ds(ofs, w)] = cmax
        cam_ref[0, pl.ds(ofs, w)] = cam
        out_ref[pl.ds(ofs, w)] = jnp.where(
            cmax < _LOW, jnp.int32(-1),
            jnp.where(cmax < _HIGH, jnp.int32(-2), cam))

    rm = rmax_ref[...]
    for k in range(_NCH):
        ofs, w, blk = chunk_blk(k)
        low = cmax_ref[0, pl.ds(ofs, w)] < _HIGH

        @pl.when(jnp.any(low))
        def _(ofs=ofs, w=w, blk=blk, low=low):
            cam = cam_ref[0, pl.ds(ofs, w)]
            upd = jnp.any(blk == rm, axis=0)
            m = jnp.where(low & jnp.logical_not(upd),
                          jnp.where(cmax_ref[0, pl.ds(ofs, w)] < _LOW,
                                    jnp.int32(-1), jnp.int32(-2)),
                          cam)
            out_ref[pl.ds(ofs, w)] = m


def kernel(match_quality_matrix):
    return pl.pallas_call(
        _body,
        out_shape=jax.ShapeDtypeStruct((_C,), jnp.int32),
        in_specs=[pl.BlockSpec(memory_space=pl.ANY)],
        out_specs=pl.BlockSpec(memory_space=pltpu.VMEM),
        scratch_shapes=[
            pltpu.VMEM((_R, _BUFW), jnp.float32),
            pltpu.VMEM((_R, _TAILW), jnp.float32),
            pltpu.VMEM((1, _C), jnp.float32),
            pltpu.VMEM((1, _C), jnp.int32),
            pltpu.VMEM((_R, 1), jnp.float32),
            pltpu.SemaphoreType.DMA((_NCH,)),
        ],
        compiler_params=pltpu.CompilerParams(
            vmem_limit_bytes=100 * 1024 * 1024,
        ),
    )(match_quality_matrix)
